# Initial kernel scaffold; baseline (speedup 1.0000x reference)
import jax
import jax.numpy as jnp
from jax.experimental import pallas as pl


def kernel(utterances, table, W, b):
    raise NotImplementedError("write your pallas kernel here")


# speedup vs baseline: 2.2776x; 2.2776x over previous
import functools

import jax
import jax.numpy as jnp
from jax import lax
from jax.experimental import pallas as pl
from jax.experimental.pallas import tpu as pltpu
from jax.experimental.pallas import tpu_sc as plsc


def _transform_body(tbl_ref, w_ref, b_ref, out_ref):
    out_ref[...] = (
        jnp.dot(tbl_ref[...], w_ref[...], preferred_element_type=jnp.float32)
        + b_ref[...]
    )


@functools.partial(jax.jit, static_argnames=("blk",))
def _transform(table, W, b, blk=2000):
    V, D = table.shape
    return pl.pallas_call(
        _transform_body,
        grid=(V // blk,),
        in_specs=[
            pl.BlockSpec((blk, D), lambda i: (i, 0)),
            pl.BlockSpec((D, D), lambda i: (0, 0)),
            pl.BlockSpec((1, D), lambda i: (0, 0)),
        ],
        out_specs=pl.BlockSpec((blk, D), lambda i: (i, 0)),
        out_shape=jax.ShapeDtypeStruct((V, D), jnp.float32),
    )(table, W, b.reshape(1, D))


CHUNK = 128


@functools.lru_cache(maxsize=None)
def _make_gather(B, D):
    info = plsc.get_sparse_core_info()
    nc, ns = info.num_cores, info.num_subcores
    nw = nc * ns
    rows_per_w = B // nw
    n_chunks = rows_per_w // CHUNK
    assert rows_per_w * nw == B and n_chunks * CHUNK == rows_per_w

    mesh = plsc.VectorSubcoreMesh(core_axis_name="c", subcore_axis_name="s")

    @functools.partial(
        pl.kernel,
        out_type=jax.ShapeDtypeStruct((B, D), jnp.float32),
        mesh=mesh,
        scratch_types=[
            pltpu.VMEM((rows_per_w,), jnp.int32),
            pltpu.VMEM((CHUNK, D), jnp.float32),
            pltpu.SemaphoreType.DMA,
        ],
    )
    def gather_k(tbl_hbm, idx_hbm, out_hbm, idx_v, buf, gsem):
        wid = lax.axis_index("s") * nc + lax.axis_index("c")
        base = wid * rows_per_w

        pltpu.sync_copy(idx_hbm.at[pl.ds(base, rows_per_w)], idx_v)

        def body(j, _):
            pltpu.async_copy(
                tbl_hbm.at[idx_v.at[pl.ds(j * CHUNK, CHUNK)]], buf, gsem
            ).wait()
            pltpu.sync_copy(buf, out_hbm.at[pl.ds(base + j * CHUNK, CHUNK)])
            return 0

        lax.fori_loop(0, n_chunks, body, 0)

    return gather_k


def kernel(utterances, table, W, b):
    batch, hist = utterances.shape
    D = table.shape[1]
    B = batch * hist

    transformed = _transform(table, W, b)
    idx = utterances.reshape(B).astype(jnp.int32)
    out = _make_gather(B, D)(transformed, idx)
    return out.reshape(batch, hist, D)


# speedup vs baseline: 2.5006x; 1.0979x over previous
import functools

import jax
import jax.numpy as jnp
from jax import lax
from jax.experimental import pallas as pl
from jax.experimental.pallas import tpu as pltpu
from jax.experimental.pallas import tpu_sc as plsc


def _transform_body(tbl_ref, w_ref, b_ref, out_ref):
    out_ref[...] = (
        jnp.dot(tbl_ref[...], w_ref[...], preferred_element_type=jnp.float32)
        + b_ref[...]
    )


@functools.partial(jax.jit, static_argnames=("blk",))
def _transform(table, W, b, blk=2000):
    V, D = table.shape
    return pl.pallas_call(
        _transform_body,
        grid=(V // blk,),
        in_specs=[
            pl.BlockSpec((blk, D), lambda i: (i, 0)),
            pl.BlockSpec((D, D), lambda i: (0, 0)),
            pl.BlockSpec((1, D), lambda i: (0, 0)),
        ],
        out_specs=pl.BlockSpec((blk, D), lambda i: (i, 0)),
        out_shape=jax.ShapeDtypeStruct((V, D), jnp.float32),
    )(table, W, b.reshape(1, D))


CHUNK = 128


@functools.lru_cache(maxsize=None)
def _make_gather(B, D):
    info = plsc.get_sparse_core_info()
    nc, ns = info.num_cores, info.num_subcores
    nw = nc * ns
    rows_per_w = B // nw
    n_chunks = rows_per_w // CHUNK
    assert rows_per_w * nw == B and n_chunks * CHUNK == rows_per_w

    mesh = plsc.VectorSubcoreMesh(core_axis_name="c", subcore_axis_name="s")

    @functools.partial(
        pl.kernel,
        out_type=jax.ShapeDtypeStruct((B, D), jnp.float32),
        mesh=mesh,
        scratch_types=[
            pltpu.VMEM((rows_per_w,), jnp.int32),
            pltpu.VMEM((CHUNK, D), jnp.float32),
            pltpu.VMEM((CHUNK, D), jnp.float32),
            pltpu.SemaphoreType.DMA,
            pltpu.SemaphoreType.DMA,
            pltpu.SemaphoreType.DMA,
            pltpu.SemaphoreType.DMA,
        ],
    )
    def gather_k(tbl_hbm, idx_hbm, out_hbm, idx_v, buf0, buf1,
                 gsem0, gsem1, wsem0, wsem1):
        wid = lax.axis_index("s") * nc + lax.axis_index("c")
        base = wid * rows_per_w

        pltpu.sync_copy(idx_hbm.at[pl.ds(base, rows_per_w)], idx_v)

        bufs = (buf0, buf1)
        gsems = (gsem0, gsem1)
        wsems = (wsem0, wsem1)

        def gather(j, p):
            return pltpu.make_async_copy(
                tbl_hbm.at[idx_v.at[pl.ds(j * CHUNK, CHUNK)]],
                bufs[p], gsems[p])

        def writeback(j, p):
            return pltpu.make_async_copy(
                bufs[p], out_hbm.at[pl.ds(base + j * CHUNK, CHUNK)],
                wsems[p])

        gather(0, 0).start()

        def body(g, _):
            for p in range(2):
                j = 2 * g + p

                @pl.when(j + 1 < n_chunks)
                def _():
                    @pl.when(j >= 1)
                    def _():
                        writeback(j - 1, 1 - p).wait()
                    gather(j + 1, 1 - p).start()

                gather(j, p).wait()
                writeback(j, p).start()
            return 0

        lax.fori_loop(0, n_chunks // 2, body, 0)

        writeback(n_chunks - 2, 0).wait()
        writeback(n_chunks - 1, 1).wait()

    return gather_k


def kernel(utterances, table, W, b):
    batch, hist = utterances.shape
    D = table.shape[1]
    B = batch * hist

    transformed = _transform(table, W, b)
    idx = utterances.reshape(B).astype(jnp.int32)
    out = _make_gather(B, D)(transformed, idx)
    return out.reshape(batch, hist, D)


# speedup vs baseline: 3.5554x; 1.4219x over previous
import functools

import jax
import jax.numpy as jnp
from jax import lax
from jax.experimental import pallas as pl
from jax.experimental.pallas import tpu as pltpu
from jax.experimental.pallas import tpu_sc as plsc


def _transform_body(tbl_ref, w_ref, b_ref, out_ref):
    out_ref[...] = (
        jnp.dot(tbl_ref[...], w_ref[...], preferred_element_type=jnp.float32)
        + b_ref[...]
    )


@functools.partial(jax.jit, static_argnames=("blk",))
def _transform(table, W, b, blk=2000):
    V, D = table.shape
    return pl.pallas_call(
        _transform_body,
        grid=(V // blk,),
        in_specs=[
            pl.BlockSpec((blk, D), lambda i: (i, 0)),
            pl.BlockSpec((D, D), lambda i: (0, 0)),
            pl.BlockSpec((1, D), lambda i: (0, 0)),
        ],
        out_specs=pl.BlockSpec((blk, D), lambda i: (i, 0)),
        out_shape=jax.ShapeDtypeStruct((V, D), jnp.float32),
    )(table, W, b.reshape(1, D))


@functools.lru_cache(maxsize=None)
def _make_gather(batch, hist, D):
    info = plsc.get_sparse_core_info()
    nc, ns = info.num_cores, info.num_subcores
    nw = nc * ns
    b_per_w = batch // nw
    assert b_per_w * nw == batch

    mesh = plsc.VectorSubcoreMesh(core_axis_name="c", subcore_axis_name="s")

    @functools.partial(
        pl.kernel,
        out_type=jax.ShapeDtypeStruct((batch, hist, D), jnp.float32),
        mesh=mesh,
        scratch_types=[
            pltpu.VMEM((b_per_w, hist), jnp.int32),
            pltpu.VMEM((hist, D), jnp.float32),
            pltpu.VMEM((hist, D), jnp.float32),
            pltpu.SemaphoreType.DMA,
            pltpu.SemaphoreType.DMA,
            pltpu.SemaphoreType.DMA,
            pltpu.SemaphoreType.DMA,
        ],
    )
    def gather_k(tbl_hbm, idx_hbm, out_hbm, idx_v, buf0, buf1,
                 gsem0, gsem1, wsem0, wsem1):
        wid = lax.axis_index("s") * nc + lax.axis_index("c")
        base = wid * b_per_w

        pltpu.sync_copy(idx_hbm.at[pl.ds(base, b_per_w)], idx_v)

        bufs = (buf0, buf1)
        gsems = (gsem0, gsem1)
        wsems = (wsem0, wsem1)

        def gather(k, p):
            return pltpu.make_async_copy(
                tbl_hbm.at[idx_v.at[k]], bufs[p], gsems[p])

        def writeback(k, p):
            return pltpu.make_async_copy(
                bufs[p], out_hbm.at[base + k], wsems[p])

        gather(0, 0).start()

        def body(g, _):
            for p in range(2):
                k = 2 * g + p

                @pl.when(k + 1 < b_per_w)
                def _():
                    @pl.when(k >= 1)
                    def _():
                        writeback(k - 1, 1 - p).wait()
                    gather(k + 1, 1 - p).start()

                gather(k, p).wait()
                writeback(k, p).start()
            return 0

        lax.fori_loop(0, b_per_w // 2, body, 0)

        writeback(b_per_w - 2, 0).wait()
        writeback(b_per_w - 1, 1).wait()

    return gather_k


def kernel(utterances, table, W, b):
    batch, hist = utterances.shape
    D = table.shape[1]

    transformed = _transform(table, W, b)
    idx = utterances.astype(jnp.int32)
    return _make_gather(batch, hist, D)(transformed, idx)
